# baseline (speedup 1.0000x reference)
import functools

import jax
import jax.numpy as jnp
from jax import lax
from jax.experimental import pallas as pl
from jax.experimental.pallas import tpu as pltpu
from jax.experimental.pallas import tpu_sc as plsc


def _make_sc_kernel(B, DZ, DS, V, b_per_w, NC):
    mesh = plsc.VectorSubcoreMesh(core_axis_name="c", subcore_axis_name="s")
    n_groups = b_per_w // 16
    c_per_w = b_per_w // 128
    ot_words = DZ * b_per_w

    @functools.partial(
        pl.kernel,
        mesh=mesh,
        out_type=(
            jax.ShapeDtypeStruct((B, DZ + DS), jnp.float32),
            jax.ShapeDtypeStruct((B * DZ,), jnp.float32),
            jax.ShapeDtypeStruct((B * DS,), jnp.float32),
        ),
        scratch_types=[
            pltpu.VMEM((b_per_w,), jnp.int32),
            pltpu.VMEM((b_per_w,), jnp.int32),
            pltpu.VMEM((V, DZ), jnp.float32),
            pltpu.VMEM((V, DS), jnp.float32),
            pltpu.VMEM((V * (DZ + 1),), jnp.float32),
            pltpu.VMEM((V * (DS + 1),), jnp.float32),
            pltpu.VMEM((b_per_w, DZ), jnp.float32),
            pltpu.VMEM((b_per_w, DS), jnp.float32),
            pltpu.VMEM((ot_words // 2,), jnp.float32),
            pltpu.VMEM((ot_words // 2,), jnp.float32),
            pltpu.SemaphoreType.DMA,
            pltpu.SemaphoreType.DMA,
            pltpu.SemaphoreType.DMA,
            pltpu.SemaphoreType.DMA,
            pltpu.SemaphoreType.DMA,
        ],
        compiler_params=pltpu.CompilerParams(
            use_tc_tiling_on_sc=False, needs_layout_passes=False,
            disable_bounds_checks=True),
    )
    def sc_kernel(zi_hbm, li_hbm, zt_hbm, st_hbm,
                  tl_hbm, zo_hbm, so_hbm,
                  zi_v, li_v, zt_v, st_v, ztp_v, stp_v, z_v, s_v, ot_a, ot_b,
                  sem_i, sem_t, sem_z, sem_s, sem_o):
        wid = lax.axis_index("s") * NC + lax.axis_index("c")
        base = wid * b_per_w
        ci1 = pltpu.async_copy(zi_hbm.at[pl.ds(base, b_per_w)], zi_v, sem_i)
        ci2 = pltpu.async_copy(li_hbm.at[pl.ds(base, b_per_w)], li_v, sem_i)
        ct1 = pltpu.async_copy(zt_hbm, zt_v, sem_t)
        ct2 = pltpu.async_copy(st_hbm, st_v, sem_t)

        def pad_stage(dense_v, padded_v, D):
            def r_body(r, _):
                for k in range(D // 16):
                    padded_v[pl.ds(r * (D + 1) + k * 16, 16)] = (
                        dense_v[r, pl.ds(k * 16, 16)])
                return _
            lax.fori_loop(0, V, r_body, 0)
        ci1.wait()
        cz = pltpu.async_copy(zt_hbm.at[zi_v], z_v, sem_z)
        ci2.wait()
        cs = pltpu.async_copy(st_hbm.at[li_v], s_v, sem_s)

        one = jnp.full((16,), 1, jnp.int32)

        def assemble(idx_v, tabp_v, D, buf, r0, nr):
            d0 = jnp.full((16,), r0 * 8, jnp.int32)

            def g_body(g, _):
                c = g // 8
                cig = g - c * 8
                goff = c * 1024 + cig * 16
                idx16 = idx_v[pl.ds(g * 16, 16)]
                fidx = idx16 * (D + 1) + d0
                for rp in range(nr):
                    vals = []
                    for k in range(8):
                        if rp * 8 + k > 0:
                            fidx = fidx + one
                        vals.append(plsc.load_gather(tabp_v, [fidx]))
                    for k in range(8):
                        buf[pl.ds(goff + rp * (c_per_w * 1024)
                                  + k * 128, 16)] = vals[k]
                return _
            lax.fori_loop(0, n_groups, g_body, 0)

        rchunk = c_per_w * 1024
        rstride = (B // 128) * 1024

        def dump(buf, dst_hbm, r0, nr):
            cps = []
            for rp in range(nr):
                cps.append(pltpu.async_copy(
                    buf.at[pl.ds(rp * rchunk, rchunk)],
                    dst_hbm.at[pl.ds((r0 + rp) * rstride + wid * rchunk,
                                     rchunk)],
                    sem_o))
            return cps

        nrz = DZ // 16
        nrs = DS // 16
        ct1.wait()
        pad_stage(zt_v, ztp_v, DZ)
        assemble(zi_v, ztp_v, DZ, ot_a, 0, nrz)
        oa = dump(ot_a, zo_hbm, 0, nrz)
        assemble(zi_v, ztp_v, DZ, ot_b, nrz, nrz)
        ob = dump(ot_b, zo_hbm, nrz, nrz)
        cz.wait()
        w1 = pltpu.async_copy(
            z_v, tl_hbm.at[pl.ds(base, b_per_w), pl.ds(0, DZ)], sem_z)
        ct2.wait()
        pad_stage(st_v, stp_v, DS)
        for cp in oa:
            cp.wait()
        assemble(li_v, stp_v, DS, ot_a, 0, nrs)
        oc = dump(ot_a, so_hbm, 0, nrs)
        for cp in ob:
            cp.wait()
        assemble(li_v, stp_v, DS, ot_b, nrs, nrs)
        od = dump(ot_b, so_hbm, nrs, nrs)
        cs.wait()
        w2 = pltpu.async_copy(
            s_v, tl_hbm.at[pl.ds(base, b_per_w), pl.ds(DZ, DS)], sem_s)
        w1.wait()
        w2.wait()
        for cp in oc + od:
            cp.wait()

    return sc_kernel


def kernel(sample_indices, batch_size, labels, z_table, s_tissue_table):
    B = sample_indices.shape[0]
    V, DZ = z_table.shape
    DS = s_tissue_table.shape[1]
    info = plsc.get_sparse_core_info()
    NC, NS = info.num_cores, info.num_subcores
    NW = NC * NS
    b_per_w = B // NW

    zi = sample_indices.astype(jnp.int32)
    li = labels[:, 0].astype(jnp.int32)

    sc_kernel = _make_sc_kernel(B, DZ, DS, V, b_per_w, NC)
    total_latent, zo, so = sc_kernel(zi, li, z_table, s_tissue_table)
    z = (zo.reshape(DZ // 8, B // 128, 8, 128)
         .transpose(1, 3, 0, 2).reshape(B, DZ))
    s = (so.reshape(DS // 8, B // 128, 8, 128)
         .transpose(1, 3, 0, 2).reshape(B, DS))
    return (total_latent, z, s)


# speedup vs baseline: 1.0148x; 1.0148x over previous
import functools

import jax
import jax.numpy as jnp
from jax import lax
from jax.experimental import pallas as pl
from jax.experimental.pallas import tpu as pltpu
from jax.experimental.pallas import tpu_sc as plsc


def _make_sc_kernel(B, DZ, DS, V, b_per_w, NC):
    mesh = plsc.VectorSubcoreMesh(core_axis_name="c", subcore_axis_name="s")
    n_groups = b_per_w // 16
    c_per_w = b_per_w // 128
    ot_words = DZ * b_per_w

    @functools.partial(
        pl.kernel,
        mesh=mesh,
        out_type=(
            jax.ShapeDtypeStruct((B, DZ + DS), jnp.float32),
            jax.ShapeDtypeStruct((B * DZ,), jnp.float32),
            jax.ShapeDtypeStruct((B * DS,), jnp.float32),
        ),
        scratch_types=[
            pltpu.VMEM((b_per_w,), jnp.int32),
            pltpu.VMEM((b_per_w,), jnp.int32),
            pltpu.VMEM((V, DZ + 1), jnp.float32),
            pltpu.VMEM((V, DS + 1), jnp.float32),
            pltpu.VMEM((b_per_w, DZ), jnp.float32),
            pltpu.VMEM((b_per_w, DS), jnp.float32),
            pltpu.VMEM((ot_words // 2,), jnp.float32),
            pltpu.VMEM((ot_words // 2,), jnp.float32),
            pltpu.SemaphoreType.DMA,
            pltpu.SemaphoreType.DMA,
            pltpu.SemaphoreType.DMA,
            pltpu.SemaphoreType.DMA,
            pltpu.SemaphoreType.DMA,
        ],
        compiler_params=pltpu.CompilerParams(
            use_tc_tiling_on_sc=False, needs_layout_passes=False,
            disable_bounds_checks=True),
    )
    def sc_kernel(zi_hbm, li_hbm, zt_hbm, st_hbm,
                  tl_hbm, zo_hbm, so_hbm,
                  zi_v, li_v, zt_v, st_v, z_v, s_v, ot_a, ot_b,
                  sem_i, sem_t, sem_z, sem_s, sem_o):
        wid = lax.axis_index("s") * NC + lax.axis_index("c")
        base = wid * b_per_w
        ci1 = pltpu.async_copy(zi_hbm.at[pl.ds(base, b_per_w)], zi_v, sem_i)
        ci2 = pltpu.async_copy(li_hbm.at[pl.ds(base, b_per_w)], li_v, sem_i)
        ct1 = pltpu.async_copy(zt_hbm, zt_v.at[:, pl.ds(0, DZ)], sem_t)
        ct2 = pltpu.async_copy(st_hbm, st_v.at[:, pl.ds(0, DS)], sem_t)
        ci1.wait()
        cz = pltpu.async_copy(zt_hbm.at[zi_v], z_v, sem_z)
        ci2.wait()
        cs = pltpu.async_copy(st_hbm.at[li_v], s_v, sem_s)

        one = jnp.full((16,), 1, jnp.int32)

        def assemble(idx_v, tab_v, buf, r0, nr):
            d0 = jnp.full((16,), r0 * 8, jnp.int32)

            def g_body(g, _):
                c = g // 8
                cig = g - c * 8
                goff = c * 1024 + cig * 16
                idx16 = idx_v[pl.ds(g * 16, 16)]
                dsplat = d0
                for rp in range(nr):
                    for rib in range(0, 8, 8):
                        vals = []
                        for k in range(8):
                            if rp * 8 + rib + k > 0:
                                dsplat = dsplat + one
                            vals.append(
                                plsc.load_gather(tab_v, [idx16, dsplat]))
                        for k in range(8):
                            ri = rib + k
                            buf[pl.ds(goff + rp * (c_per_w * 1024)
                                      + ri * 128, 16)] = vals[k]
                return _
            lax.fori_loop(0, n_groups, g_body, 0)

        rchunk = c_per_w * 1024
        rstride = (B // 128) * 1024

        def dump(buf, dst_hbm, r0, nr):
            cps = []
            for rp in range(nr):
                cps.append(pltpu.async_copy(
                    buf.at[pl.ds(rp * rchunk, rchunk)],
                    dst_hbm.at[pl.ds((r0 + rp) * rstride + wid * rchunk,
                                     rchunk)],
                    sem_o))
            return cps

        nrz = DZ // 16
        nrs = DS // 16
        ct1.wait()
        assemble(zi_v, zt_v, ot_a, 0, nrz)
        oa = dump(ot_a, zo_hbm, 0, nrz)
        assemble(zi_v, zt_v, ot_b, nrz, nrz)
        ob = dump(ot_b, zo_hbm, nrz, nrz)
        cz.wait()
        w1 = pltpu.async_copy(
            z_v, tl_hbm.at[pl.ds(base, b_per_w), pl.ds(0, DZ)], sem_z)
        ct2.wait()
        for cp in oa:
            cp.wait()
        assemble(li_v, st_v, ot_a, 0, nrs)
        oc = dump(ot_a, so_hbm, 0, nrs)
        for cp in ob:
            cp.wait()
        assemble(li_v, st_v, ot_b, nrs, nrs)
        od = dump(ot_b, so_hbm, nrs, nrs)
        cs.wait()
        w2 = pltpu.async_copy(
            s_v, tl_hbm.at[pl.ds(base, b_per_w), pl.ds(DZ, DS)], sem_s)
        w1.wait()
        w2.wait()
        for cp in oc + od:
            cp.wait()

    return sc_kernel


def kernel(sample_indices, batch_size, labels, z_table, s_tissue_table):
    B = sample_indices.shape[0]
    V, DZ = z_table.shape
    DS = s_tissue_table.shape[1]
    info = plsc.get_sparse_core_info()
    NC, NS = info.num_cores, info.num_subcores
    NW = NC * NS
    b_per_w = B // NW

    zi = sample_indices.astype(jnp.int32)
    li = labels[:, 0].astype(jnp.int32)

    sc_kernel = _make_sc_kernel(B, DZ, DS, V, b_per_w, NC)
    total_latent, zo, so = sc_kernel(zi, li, z_table, s_tissue_table)
    z = (zo.reshape(DZ // 8, B // 128, 8, 128)
         .transpose(1, 3, 0, 2).reshape(B, DZ))
    s = (so.reshape(DS // 8, B // 128, 8, 128)
         .transpose(1, 3, 0, 2).reshape(B, DS))
    return (total_latent, z, s)
